# baseline (speedup 1.0000x reference)
import functools

import jax
import jax.numpy as jnp
from jax import lax
from jax.experimental import pallas as pl
from jax.experimental.pallas import tpu as pltpu
from jax.experimental.pallas import tpu_sc as plsc

FEAT_DIM = 256
NUM_FEAT = 16384
NUM_GROUPS = 4096
MAX_FEAT_PLUS = 64

NC = 2
NS = 16
L = 16
NW = NC * NS
GPW = NUM_GROUPS // NW
D = FEAT_DIM
FP = MAX_FEAT_PLUS
NCH = D // L
ZROW = NUM_FEAT


def _grouper_body(table_hbm, idx_hbm, out_hbm, idx_v, rows_v, out_stage, sem):
    wid = lax.axis_index("s") * NC + lax.axis_index("c")
    g0 = wid * GPW
    pltpu.sync_copy(idx_hbm.at[pl.ds(g0, GPW)], idx_v)

    def group_body(g, carry):
        pltpu.async_copy(table_hbm.at[idx_v.at[g]], rows_v, sem).wait()

        def row_body(j, acc):
            return tuple(acc[c] + rows_v[j, pl.ds(c * L, L)] for c in range(NCH))

        zeros = tuple(jnp.zeros((L,), jnp.float32) for _ in range(NCH))
        acc = lax.fori_loop(0, FP, row_body, zeros)
        for c in range(NCH):
            out_stage[g, pl.ds(c * L, L)] = acc[c]
        return carry

    lax.fori_loop(0, GPW, group_body, 0)
    pltpu.sync_copy(out_stage, out_hbm.at[pl.ds(g0, GPW)])


_SCRATCH = [
    pltpu.VMEM((GPW, FP), jnp.int32),
    pltpu.VMEM((FP, D), jnp.float32),
    pltpu.VMEM((GPW, D), jnp.float32),
    pltpu.SemaphoreType.DMA,
]


@functools.lru_cache(maxsize=None)
def _grouper_sc():
    mesh = plsc.VectorSubcoreMesh(
        core_axis_name="c", subcore_axis_name="s",
        num_cores=NC, num_subcores=NS)
    return pl.kernel(
        _grouper_body,
        out_type=jax.ShapeDtypeStruct((NUM_GROUPS, D), jnp.float32),
        mesh=mesh,
        scratch_types=_SCRATCH,
    )


@jax.jit
def kernel(in_features, W, grp_edge_feat, edge_to_node, grp_edge_idx_plus,
           grp_num_feat, grp_feat_idx_plus):
    ratio = 1.0 / grp_num_feat.astype(jnp.float32)
    csum = jnp.cumsum(
        jnp.broadcast_to(ratio[:, None], (NUM_GROUPS, FP)), axis=1)
    hard = csum <= 1.0
    idx_m = jnp.where(hard, grp_feat_idx_plus, ZROW).astype(jnp.int32)
    table_ext = jnp.concatenate(
        [in_features, jnp.zeros((8, D), jnp.float32)], axis=0)
    return _grouper_sc()(table_ext, idx_m)
